# baseline (speedup 1.0000x reference)
import dataclasses
import functools

import jax
import jax.numpy as jnp
from jax import lax
from jax.experimental import pallas as pl
from jax.experimental.pallas import tpu as pltpu
from jax.experimental.pallas import tpu_sc as plsc

N_CORES = 2
N_SUBCORES = 16
N_TILES = N_CORES * N_SUBCORES
LANES = 16
WINDOW = 80
NBUF = 2
NACC = 8


def _to_packed_bf16_tc(table, rel=None):
    def body(t_ref, o_ref):
        o_ref[...] = t_ref[...].astype(jnp.bfloat16)

    def body_scaled(t_ref, r_ref, o_ref):
        o_ref[...] = (t_ref[...] * r_ref[...]).astype(jnp.bfloat16)

    out_shape = jax.ShapeDtypeStruct(table.shape, jnp.bfloat16)
    if rel is None:
        bf = pl.pallas_call(body, out_shape=out_shape)(table)
    else:
        bf = pl.pallas_call(body_scaled, out_shape=out_shape)(table, rel)
    n, d = table.shape
    return lax.bitcast_convert_type(
        bf.reshape(n, d // 2, 2), jnp.int32)


@functools.lru_cache(maxsize=2)
def _make_sc_scorer(n_edges, dim):
    assert n_edges % N_TILES == 0
    n_per_tile = n_edges // N_TILES
    assert n_per_tile % (WINDOW * NBUF) == 0
    n_windows = n_per_tile // WINDOW
    n_groups = WINDOW // LANES
    words = dim // 2
    assert words & (words - 1) == 0

    mesh = plsc.VectorSubcoreMesh(core_axis_name="c", subcore_axis_name="s")

    cp = pltpu.CompilerParams()
    if "needs_layout_passes" in pltpu.CompilerParams.__dataclass_fields__:
        cp = dataclasses.replace(cp, needs_layout_passes=False)
    if "use_tc_tiling_on_sc" in pltpu.CompilerParams.__dataclass_fields__:
        cp = dataclasses.replace(cp, use_tc_tiling_on_sc=False)

    @functools.partial(
        pl.kernel,
        compiler_params=cp,
        out_type=jax.ShapeDtypeStruct((n_edges,), jnp.float32),
        mesh=mesh,
        scratch_types=[
            pltpu.VMEM((n_per_tile,), jnp.int32),
            pltpu.VMEM((n_per_tile,), jnp.int32),
            pltpu.VMEM((NBUF, WINDOW, words), jnp.int32),
            pltpu.VMEM((NBUF, WINDOW, words), jnp.int32),
            pltpu.VMEM((NBUF, WINDOW), jnp.float32),
            pltpu.SemaphoreType.DMA,
            pltpu.SemaphoreType.DMA,
            pltpu.SemaphoreType.DMA,
            pltpu.SemaphoreType.DMA,
            pltpu.SemaphoreType.DMA,
        ],
    )
    def scorer(src_hbm, dst_hbm, user_hbm, item_hbm, out_hbm,
               src_v, dst_v, h_v, t_v, o_v, sem_i, sem_g0, sem_g1,
               sem_o0, sem_o1):
        wid = lax.axis_index("s") * N_CORES + lax.axis_index("c")
        tile_base = wid * n_per_tile
        sem_g = (sem_g0, sem_g1)
        sem_o = (sem_o0, sem_o1)

        ci0 = pltpu.async_copy(
            src_hbm.at[pl.ds(tile_base, n_per_tile)], src_v, sem_i)
        ci1 = pltpu.async_copy(
            dst_hbm.at[pl.ds(tile_base, n_per_tile)], dst_v, sem_i)
        ci0.wait()
        ci1.wait()

        def g_copies(w, b):
            sl = pl.ds(w * WINDOW, WINDOW)
            return (
                pltpu.make_async_copy(
                    user_hbm.at[src_v.at[sl]], h_v.at[b], sem_g[b]),
                pltpu.make_async_copy(
                    item_hbm.at[dst_v.at[sl]], t_v.at[b], sem_g[b]),
            )

        def o_copy(w, b):
            return pltpu.make_async_copy(
                o_v.at[b], out_hbm.at[pl.ds(tile_base + w * WINDOW, WINDOW)],
                sem_o[b])

        for b in range(NBUF):
            for c in g_copies(b, b):
                c.start()

        @pl.loop(0, n_windows, step=NBUF)
        def _(win):
            for b in range(NBUF):
                w = win + b

                @pl.when(w < NBUF)
                def _():
                    for c in g_copies(w, b):
                        c.wait()

                @pl.when(w >= NBUF)
                def _():
                    o_copy(w - NBUF, b).wait()

                hb = h_v.at[b]
                tb = t_v.at[b]

                @pl.loop(0, n_groups)
                def _(g):
                    lane = lax.iota(jnp.int32, LANES)
                    rows = g * LANES + lane
                    accs = [jnp.zeros((LANES,), jnp.float32)
                            for _ in range(NACC)]
                    for c in range(words):
                        col = lane ^ c
                        hw = plsc.load_gather(hb, [rows, col])
                        tw = plsc.load_gather(tb, [rows, col])
                        prod = (plsc.bitcast(hw, jnp.bfloat16)
                                * plsc.bitcast(tw, jnp.bfloat16))
                        pa, pb = plsc.unpack(
                            prod,
                            format=plsc.PackFormat.INTERLEAVED,
                            preferred_element_type=jnp.float32)
                        accs[(2 * c) % NACC] = accs[(2 * c) % NACC] + pa
                        accs[(2 * c + 1) % NACC] = (
                            accs[(2 * c + 1) % NACC] + pb)
                    while len(accs) > 1:
                        accs = [x + y for x, y in zip(accs[::2], accs[1::2])]
                    o_v.at[b][pl.ds(g * LANES, LANES)] = accs[0]

                o_copy(w, b).start()


        for b in range(NBUF):
            o_copy(n_windows - NBUF + b, b).wait()

    return scorer


@jax.jit
def kernel(edge_pos, edge_neg, emb_user, emb_item, relation_embedding):
    e = edge_pos.shape[1]
    src = jnp.concatenate([edge_pos[0], edge_neg[0]]).astype(jnp.int32)
    dst = jnp.concatenate([edge_pos[1], edge_neg[1]]).astype(jnp.int32)
    user_packed = _to_packed_bf16_tc(emb_user)
    item_packed = _to_packed_bf16_tc(emb_item, relation_embedding)
    scorer = _make_sc_scorer(2 * e, emb_user.shape[1])
    scores = scorer(src, dst, user_packed, item_packed)
    return scores[:e], scores[e:]


# speedup vs baseline: 2.4315x; 2.4315x over previous
import dataclasses
import functools

import jax
import jax.numpy as jnp
from jax import lax
from jax.experimental import pallas as pl
from jax.experimental.pallas import tpu as pltpu
from jax.experimental.pallas import tpu_sc as plsc

N_CORES = 2
N_SUBCORES = 16
N_TILES = N_CORES * N_SUBCORES
LANES = 16
WINDOW = 160
NBUF = 2
NACC = 8
NILV = 2


def _to_packed_bf16_tc(table, rel=None):
    def body(t_ref, o_ref):
        o_ref[...] = t_ref[...].astype(jnp.bfloat16)

    def body_scaled(t_ref, r_ref, o_ref):
        o_ref[...] = (t_ref[...] * r_ref[...]).astype(jnp.bfloat16)

    out_shape = jax.ShapeDtypeStruct(table.shape, jnp.bfloat16)
    if rel is None:
        bf = pl.pallas_call(body, out_shape=out_shape)(table)
    else:
        bf = pl.pallas_call(body_scaled, out_shape=out_shape)(table, rel)
    n, d = table.shape
    return lax.bitcast_convert_type(
        bf.reshape(n, d // 2, 2), jnp.int32)


@functools.lru_cache(maxsize=2)
def _make_sc_scorer(n_edges, dim):
    assert n_edges % N_TILES == 0
    n_per_tile = n_edges // N_TILES
    assert n_per_tile % WINDOW == 0
    n_windows = n_per_tile // WINDOW
    n_main = n_windows - n_windows % NBUF
    n_groups = WINDOW // LANES
    words = dim // 2
    assert words & (words - 1) == 0

    mesh = plsc.VectorSubcoreMesh(core_axis_name="c", subcore_axis_name="s")

    cp = pltpu.CompilerParams()
    if "needs_layout_passes" in pltpu.CompilerParams.__dataclass_fields__:
        cp = dataclasses.replace(cp, needs_layout_passes=False)
    if "use_tc_tiling_on_sc" in pltpu.CompilerParams.__dataclass_fields__:
        cp = dataclasses.replace(cp, use_tc_tiling_on_sc=False)

    @functools.partial(
        pl.kernel,
        compiler_params=cp,
        out_type=jax.ShapeDtypeStruct((n_edges,), jnp.float32),
        mesh=mesh,
        scratch_types=[
            pltpu.VMEM((n_per_tile,), jnp.int32),
            pltpu.VMEM((n_per_tile,), jnp.int32),
            pltpu.VMEM((NBUF, WINDOW, words), jnp.int32),
            pltpu.VMEM((NBUF, WINDOW, words), jnp.int32),
            pltpu.VMEM((NBUF, WINDOW), jnp.float32),
            pltpu.SemaphoreType.DMA,
            pltpu.SemaphoreType.DMA,
            pltpu.SemaphoreType.DMA,
            pltpu.SemaphoreType.DMA,
            pltpu.SemaphoreType.DMA,
        ],
    )
    def scorer(src_hbm, dst_hbm, user_hbm, item_hbm, out_hbm,
               src_v, dst_v, h_v, t_v, o_v, sem_i, sem_g0, sem_g1,
               sem_o0, sem_o1):
        wid = lax.axis_index("s") * N_CORES + lax.axis_index("c")
        tile_base = wid * n_per_tile
        sem_g = (sem_g0, sem_g1)
        sem_o = (sem_o0, sem_o1)

        ci0 = pltpu.async_copy(
            src_hbm.at[pl.ds(tile_base, n_per_tile)], src_v, sem_i)
        ci1 = pltpu.async_copy(
            dst_hbm.at[pl.ds(tile_base, n_per_tile)], dst_v, sem_i)
        ci0.wait()
        ci1.wait()

        def g_copies(w, b):
            cps = []
            half = WINDOW // 2
            for k in range(2):
                sl = pl.ds(w * WINDOW + k * half, half)
                dsl = pl.ds(k * half, half)
                cps.append(pltpu.make_async_copy(
                    user_hbm.at[src_v.at[sl]], h_v.at[b].at[dsl], sem_g[b]))
                cps.append(pltpu.make_async_copy(
                    item_hbm.at[dst_v.at[sl]], t_v.at[b].at[dsl], sem_g[b]))
            return cps

        def o_copy(w, b):
            return pltpu.make_async_copy(
                o_v.at[b], out_hbm.at[pl.ds(tile_base + w * WINDOW, WINDOW)],
                sem_o[b])

        def compute_window(b):
                hb = h_v.at[b]
                tb = t_v.at[b]

                @pl.loop(0, n_groups, step=NILV)
                def _(g):
                    lane = lax.iota(jnp.int32, LANES)
                    rows = [(g + i) * LANES + lane for i in range(NILV)]
                    accs = [[jnp.zeros((LANES,), jnp.float32)
                             for _ in range(NACC)] for _ in range(NILV)]
                    for c in range(words):
                        col = lane ^ c
                        for i in range(NILV):
                            hw = plsc.load_gather(hb, [rows[i], col])
                            tw = plsc.load_gather(tb, [rows[i], col])
                            prod = (plsc.bitcast(hw, jnp.bfloat16)
                                    * plsc.bitcast(tw, jnp.bfloat16))
                            pa, pb = plsc.unpack(
                                prod,
                                format=plsc.PackFormat.INTERLEAVED,
                                preferred_element_type=jnp.float32)
                            accs[i][(2 * c) % NACC] = (
                                accs[i][(2 * c) % NACC] + pa)
                            accs[i][(2 * c + 1) % NACC] = (
                                accs[i][(2 * c + 1) % NACC] + pb)
                    for i in range(NILV):
                        a = accs[i]
                        while len(a) > 1:
                            a = [x + y for x, y in zip(a[::2], a[1::2])]
                        o_v.at[b][pl.ds((g + i) * LANES, LANES)] = a[0]

        for b in range(NBUF):
            for c in g_copies(b, b):
                c.start()

        @pl.loop(0, n_main, step=NBUF)
        def _(win):
            for b in range(NBUF):
                w = win + b
                for c in g_copies(w, b):
                    c.wait()

                @pl.when(w >= NBUF)
                def _():
                    o_copy(w - NBUF, b).wait()

                compute_window(b)

                o_copy(w, b).start()

                @pl.when(w + NBUF < n_windows)
                def _():
                    for c in g_copies(w + NBUF, b):
                        c.start()

        for w in range(n_main, n_windows):
            b = w % NBUF
            for c in g_copies(w, b):
                c.wait()
            o_copy(w - NBUF, b).wait()
            compute_window(b)
            o_copy(w, b).start()

        for w in range(n_windows - NBUF, n_windows):
            o_copy(w, w % NBUF).wait()

    return scorer


@jax.jit
def kernel(edge_pos, edge_neg, emb_user, emb_item, relation_embedding):
    e = edge_pos.shape[1]
    src = jnp.concatenate([edge_pos[0], edge_neg[0]]).astype(jnp.int32)
    dst = jnp.concatenate([edge_pos[1], edge_neg[1]]).astype(jnp.int32)
    user_packed = _to_packed_bf16_tc(emb_user)
    item_packed = _to_packed_bf16_tc(emb_item, relation_embedding)
    scorer = _make_sc_scorer(2 * e, emb_user.shape[1])
    scores = scorer(src, dst, user_packed, item_packed)
    return scores[:e], scores[e:]
